# Initial kernel scaffold; baseline (speedup 1.0000x reference)
import jax
import jax.numpy as jnp
from jax.experimental import pallas as pl


def kernel(x, edge_index, edge_attr, batch, lin_W, lin_b, msg_W1, msg_b1, msg_W2, msg_b2, upd_W1, upd_b1, upd_W2, upd_b2):
    raise NotImplementedError("write your pallas kernel here")


# speedup vs baseline: 1.5445x; 1.5445x over previous
import functools

import jax
import jax.numpy as jnp
from jax import lax
from jax.experimental import pallas as pl
from jax.experimental.pallas import tpu as pltpu
from jax.experimental.pallas import tpu_sc as plsc

L = 4
N = 10000
E = 160000
EMB = 128
ED = 4
G = 16

NC = 2
NS = 16
NW = NC * NS
CW = 128
CHUNKS = 40
E_PAD = NW * CHUNKS * CW
BLK = 512
N_PAD = 10240
ROWS_PT = N_PAD // NS
NBLK = N_PAD // BLK
EBLK = E_PAD // BLK

_mesh = plsc.VectorSubcoreMesh(core_axis_name="c", subcore_axis_name="s")


@functools.partial(
    pl.kernel,
    out_type=(jax.ShapeDtypeStruct((E_PAD, EMB), jnp.float32),
              jax.ShapeDtypeStruct((E_PAD, EMB), jnp.float32)),
    mesh=_mesh,
    scratch_types=[
        pltpu.VMEM((CHUNKS, CW), jnp.int32),
        pltpu.VMEM((CHUNKS, CW), jnp.int32),
        pltpu.VMEM((CW, EMB), jnp.float32),
        pltpu.VMEM((CW, EMB), jnp.float32),
        pltpu.SemaphoreType.DMA,
        pltpu.SemaphoreType.DMA,
    ],
)
def _sc_gather(a_hbm, b_hbm, dst_hbm, src_hbm, p1_hbm, p2_hbm,
               dv, sv, buf_a, buf_b, sem_a, sem_b):
    wid = lax.axis_index("s") * NC + lax.axis_index("c")
    pltpu.sync_copy(dst_hbm.at[wid], dv)
    pltpu.sync_copy(src_hbm.at[wid], sv)
    ebase = wid * (CHUNKS * CW)

    def body(j, carry):
        ca = pltpu.async_copy(a_hbm.at[dv.at[j]], buf_a, sem_a)
        cb = pltpu.async_copy(b_hbm.at[sv.at[j]], buf_b, sem_b)
        ca.wait()
        cb.wait()
        off = ebase + j * CW
        pltpu.sync_copy(buf_a, p1_hbm.at[pl.ds(off, CW)])
        pltpu.sync_copy(buf_b, p2_hbm.at[pl.ds(off, CW)])
        return carry

    lax.fori_loop(0, CHUNKS, body, 0)


@functools.partial(
    pl.kernel,
    out_type=jax.ShapeDtypeStruct((NC, N_PAD, EMB), jnp.float32),
    mesh=_mesh,
    scratch_types=[
        pltpu.VMEM((CHUNKS, CW), jnp.int32),
        pltpu.VMEM((CW, EMB), jnp.float32),
        pltpu.VMEM_SHARED((N_PAD, EMB), jnp.float32),
    ],
)
def _sc_scatter(m_hbm, dst_hbm, z_hbm, out_hbm, dv, buf, acc):
    cid = lax.axis_index("c")
    sid = lax.axis_index("s")
    wid = sid * NC + cid
    rbase = sid * ROWS_PT
    pltpu.sync_copy(z_hbm, acc.at[pl.ds(rbase, ROWS_PT)])
    plsc.subcore_barrier()
    pltpu.sync_copy(dst_hbm.at[wid], dv)
    ebase = wid * (CHUNKS * CW)

    def body(j, carry):
        pltpu.sync_copy(m_hbm.at[pl.ds(ebase + j * CW, CW)], buf)
        pltpu.sync_copy(buf, acc.at[dv.at[j]], add=True)
        return carry

    lax.fori_loop(0, CHUNKS, body, 0)
    plsc.subcore_barrier()
    pltpu.sync_copy(acc.at[pl.ds(rbase, ROWS_PT)],
                    out_hbm.at[cid, pl.ds(rbase, ROWS_PT)])


def _lin_body(x_ref, w_ref, b_ref, o_ref):
    o_ref[...] = jnp.dot(x_ref[...], w_ref[...],
                         preferred_element_type=jnp.float32) + b_ref[...]


def _tc_lin(x, w, b):
    return pl.pallas_call(
        _lin_body,
        grid=(NBLK,),
        in_specs=[pl.BlockSpec((BLK, EMB), lambda i: (i, 0)),
                  pl.BlockSpec((EMB, EMB), lambda i: (0, 0)),
                  pl.BlockSpec((1, EMB), lambda i: (0, 0))],
        out_specs=pl.BlockSpec((BLK, EMB), lambda i: (i, 0)),
        out_shape=jax.ShapeDtypeStruct((N_PAD, EMB), jnp.float32),
    )(x, w, b)


def _prep_body(h_ref, wa_ref, wb_ref, a_ref, b_ref):
    h = h_ref[...]
    a_ref[...] = jnp.dot(h, wa_ref[...], preferred_element_type=jnp.float32)
    b_ref[...] = jnp.dot(h, wb_ref[...], preferred_element_type=jnp.float32)


def _tc_prep(h, wa, wb):
    return pl.pallas_call(
        _prep_body,
        grid=(NBLK,),
        in_specs=[pl.BlockSpec((BLK, EMB), lambda i: (i, 0)),
                  pl.BlockSpec((EMB, EMB), lambda i: (0, 0)),
                  pl.BlockSpec((EMB, EMB), lambda i: (0, 0))],
        out_specs=[pl.BlockSpec((BLK, EMB), lambda i: (i, 0)),
                   pl.BlockSpec((BLK, EMB), lambda i: (i, 0))],
        out_shape=[jax.ShapeDtypeStruct((N_PAD, EMB), jnp.float32),
                   jax.ShapeDtypeStruct((N_PAD, EMB), jnp.float32)],
    )(h, wa, wb)


def _edge_body(p1_ref, p2_ref, ea_ref, wc_ref, b1_ref, w2_ref, b2_ref, m_ref):
    t = (p1_ref[...] + p2_ref[...]
         + jnp.dot(ea_ref[...], wc_ref[...], preferred_element_type=jnp.float32)
         + b1_ref[...])
    t = jnp.tanh(t)
    m_ref[...] = jnp.tanh(
        jnp.dot(t, w2_ref[...], preferred_element_type=jnp.float32) + b2_ref[...])


def _tc_edge(p1, p2, ea, wc, b1, w2, b2):
    return pl.pallas_call(
        _edge_body,
        grid=(EBLK,),
        in_specs=[pl.BlockSpec((BLK, EMB), lambda i: (i, 0)),
                  pl.BlockSpec((BLK, EMB), lambda i: (i, 0)),
                  pl.BlockSpec((BLK, ED), lambda i: (i, 0)),
                  pl.BlockSpec((ED, EMB), lambda i: (0, 0)),
                  pl.BlockSpec((1, EMB), lambda i: (0, 0)),
                  pl.BlockSpec((EMB, EMB), lambda i: (0, 0)),
                  pl.BlockSpec((1, EMB), lambda i: (0, 0))],
        out_specs=pl.BlockSpec((BLK, EMB), lambda i: (i, 0)),
        out_shape=jax.ShapeDtypeStruct((E_PAD, EMB), jnp.float32),
    )(p1, p2, ea, wc, b1, w2, b2)


def _upd_body(h_ref, par_ref, ua_ref, ub_ref, b1_ref, w2_ref, b2_ref, o_ref):
    h = h_ref[...]
    aggr = par_ref[0] + par_ref[1]
    t = jnp.tanh(jnp.dot(h, ua_ref[...], preferred_element_type=jnp.float32)
                 + jnp.dot(aggr, ub_ref[...], preferred_element_type=jnp.float32)
                 + b1_ref[...])
    u = jnp.tanh(jnp.dot(t, w2_ref[...], preferred_element_type=jnp.float32)
                 + b2_ref[...])
    o_ref[...] = h + u


def _tc_update(h, par, ua, ub, b1, w2, b2):
    return pl.pallas_call(
        _upd_body,
        grid=(NBLK,),
        in_specs=[pl.BlockSpec((BLK, EMB), lambda i: (i, 0)),
                  pl.BlockSpec((NC, BLK, EMB), lambda i: (0, i, 0)),
                  pl.BlockSpec((EMB, EMB), lambda i: (0, 0)),
                  pl.BlockSpec((EMB, EMB), lambda i: (0, 0)),
                  pl.BlockSpec((1, EMB), lambda i: (0, 0)),
                  pl.BlockSpec((EMB, EMB), lambda i: (0, 0)),
                  pl.BlockSpec((1, EMB), lambda i: (0, 0))],
        out_specs=pl.BlockSpec((BLK, EMB), lambda i: (i, 0)),
        out_shape=jax.ShapeDtypeStruct((N_PAD, EMB), jnp.float32),
    )(h, par, ua, ub, b1, w2, b2)


def _pool_body(h_ref, bat_ref, o_ref, sums, counts):
    i = pl.program_id(0)

    @pl.when(i == 0)
    def _():
        sums[...] = jnp.zeros_like(sums)
        counts[...] = jnp.zeros_like(counts)

    bat = bat_ref[0]
    gids = lax.broadcasted_iota(jnp.int32, (G, BLK), 0)
    onehot = jnp.where(bat == gids, 1.0, 0.0)
    sums[...] += jnp.dot(onehot, h_ref[...], preferred_element_type=jnp.float32)
    counts[...] += jnp.sum(onehot, axis=1, keepdims=True)

    @pl.when(i == NBLK - 1)
    def _():
        o_ref[...] = sums[...] / jnp.maximum(counts[...], 1.0)


def _tc_pool(h, bat3):
    return pl.pallas_call(
        _pool_body,
        grid=(NBLK,),
        in_specs=[pl.BlockSpec((BLK, EMB), lambda i: (i, 0)),
                  pl.BlockSpec((1, 1, BLK), lambda i: (i, 0, 0))],
        out_specs=pl.BlockSpec((G, EMB), lambda i: (0, 0)),
        out_shape=jax.ShapeDtypeStruct((G, EMB), jnp.float32),
        scratch_shapes=[pltpu.VMEM((G, EMB), jnp.float32),
                        pltpu.VMEM((G, 1), jnp.float32)],
    )(h, bat3)


@jax.jit
def _impl(x, edge_index, edge_attr, batch, lin_W, lin_b,
          msg_W1, msg_b1, msg_W2, msg_b2,
          upd_W1, upd_b1, upd_W2, upd_b2):
    f32 = jnp.float32
    x_p = jnp.zeros((N_PAD, EMB), f32).at[:N].set(x)
    src = edge_index[0]
    dst = edge_index[1]
    src_p = jnp.zeros((E_PAD,), jnp.int32).at[:E].set(src).reshape(NW, CHUNKS, CW)
    dst_p = jnp.full((E_PAD,), N, jnp.int32).at[:E].set(dst).reshape(NW, CHUNKS, CW)
    ea_p = jnp.zeros((E_PAD, ED), f32).at[:E].set(edge_attr)
    bat3 = jnp.full((N_PAD,), -1, jnp.int32).at[:N].set(batch).reshape(NBLK, 1, BLK)
    zrows = jnp.zeros((ROWS_PT, EMB), f32)

    h = _tc_lin(x_p, lin_W, lin_b.reshape(1, EMB))
    for l in range(L):
        w1a = msg_W1[l, :EMB]
        w1b = msg_W1[l, EMB:2 * EMB]
        w1c = msg_W1[l, 2 * EMB:]
        a_tab, b_tab = _tc_prep(h, w1a, w1b)
        p1, p2 = _sc_gather(a_tab, b_tab, dst_p, src_p)
        m = _tc_edge(p1, p2, ea_p, w1c, msg_b1[l].reshape(1, EMB),
                     msg_W2[l], msg_b2[l].reshape(1, EMB))
        par = _sc_scatter(m, dst_p, zrows)
        h = _tc_update(h, par, upd_W1[l, :EMB], upd_W1[l, EMB:],
                       upd_b1[l].reshape(1, EMB), upd_W2[l],
                       upd_b2[l].reshape(1, EMB))
    return _tc_pool(h, bat3)


def kernel(x, edge_index, edge_attr, batch, lin_W, lin_b,
           msg_W1, msg_b1, msg_W2, msg_b2,
           upd_W1, upd_b1, upd_W2, upd_b2):
    return _impl(x, edge_index, edge_attr, batch, lin_W, lin_b,
                 msg_W1, msg_b1, msg_W2, msg_b2,
                 upd_W1, upd_b1, upd_W2, upd_b2)


# speedup vs baseline: 1.7638x; 1.1420x over previous
import functools

import jax
import jax.numpy as jnp
from jax import lax
from jax.experimental import pallas as pl
from jax.experimental.pallas import tpu as pltpu
from jax.experimental.pallas import tpu_sc as plsc

L = 4
N = 10000
E = 160000
EMB = 128
ED = 4
G = 16

NC = 2
NS = 16
NW = NC * NS
CW = 128
CHUNKS = 40
E_PAD = NW * CHUNKS * CW
BLK = 512
N_PAD = 10240
ROWS_PT = N_PAD // NS
NBLK = N_PAD // BLK
EBLK = E_PAD // BLK

_mesh = plsc.VectorSubcoreMesh(core_axis_name="c", subcore_axis_name="s")


GK = 1
GNB = 6
TCHUNKS = E_PAD // NS // CW
BATCHES = TCHUNKS // GK
BROWS = GK * CW


def _gather_pipe(tab_hbm, idx_hbm, out_hbm, iv, bufs, gsem, osem, tid):
    pltpu.sync_copy(idx_hbm.at[tid], iv)
    ebase = tid * (TCHUNKS * CW)

    def body(i, carry):
        for s in range(GNB):
            b = i * GNB + s
            b_o = b - (GNB - 1)
            so = (s + 1) % GNB

            @pl.when(b < BATCHES)
            def _():
                @pl.when(b >= GNB)
                def _():
                    pltpu.make_async_copy(
                        bufs.at[s], out_hbm.at[pl.ds(0, BROWS)], osem[s]).wait()
                for k in range(GK):
                    pltpu.async_copy(tab_hbm.at[iv.at[b * GK + k]],
                                     bufs.at[s, pl.ds(k * CW, CW)], gsem[s])

            @pl.when((b_o >= 0) & (b_o < BATCHES))
            def _():
                for k in range(GK):
                    pltpu.make_async_copy(
                        tab_hbm.at[iv.at[b_o * GK + k]],
                        bufs.at[so, pl.ds(k * CW, CW)], gsem[so]).wait()
                off = ebase + b_o * BROWS
                pltpu.async_copy(bufs.at[so], out_hbm.at[pl.ds(off, BROWS)],
                                 osem[so])
        return carry

    lax.fori_loop(0, BATCHES // GNB + 2, body, 0)
    for s in range(GNB):
        pltpu.make_async_copy(bufs.at[s], out_hbm.at[pl.ds(0, BROWS)],
                              osem[s]).wait()


@functools.partial(
    pl.kernel,
    out_type=(jax.ShapeDtypeStruct((E_PAD, EMB), jnp.float32),
              jax.ShapeDtypeStruct((E_PAD, EMB), jnp.float32)),
    mesh=_mesh,
    scratch_types=[
        pltpu.VMEM((TCHUNKS, CW), jnp.int32),
        pltpu.VMEM((GNB, BROWS, EMB), jnp.float32),
    ] + [pltpu.SemaphoreType.DMA] * (2 * GNB),
)
def _sc_gather(a_hbm, b_hbm, dstg_hbm, srcg_hbm, p1_hbm, p2_hbm,
               iv, bufs, *sems):
    gsem = sems[0:GNB]
    osem = sems[GNB:2 * GNB]
    cid = lax.axis_index("c")
    tid = lax.axis_index("s")

    @pl.when(cid == 0)
    def _():
        _gather_pipe(a_hbm, dstg_hbm, p1_hbm, iv, bufs, gsem, osem, tid)

    @pl.when(cid == 1)
    def _():
        _gather_pipe(b_hbm, srcg_hbm, p2_hbm, iv, bufs, gsem, osem, tid)


SNBUF = 2


@functools.partial(
    pl.kernel,
    out_type=jax.ShapeDtypeStruct((NC, N_PAD, EMB), jnp.float32),
    mesh=_mesh,
    scratch_types=[
        pltpu.VMEM((CHUNKS, CW), jnp.int32),
        pltpu.VMEM((SNBUF, CW, EMB), jnp.float32),
        pltpu.VMEM_SHARED((N_PAD, EMB), jnp.float32),
    ] + [pltpu.SemaphoreType.DMA] * (2 * SNBUF),
)
def _sc_scatter(m_hbm, dst_hbm, z_hbm, out_hbm, dv, buf, acc, *sems):
    lsem = sems[0:SNBUF]
    asem = sems[SNBUF:2 * SNBUF]
    cid = lax.axis_index("c")
    sid = lax.axis_index("s")
    wid = sid * NC + cid
    rbase = sid * ROWS_PT
    pltpu.sync_copy(z_hbm, acc.at[pl.ds(rbase, ROWS_PT)])
    pltpu.sync_copy(dst_hbm.at[wid], dv)
    plsc.subcore_barrier()
    ebase = wid * (CHUNKS * CW)

    def body(i, carry):
        for s in range(SNBUF):
            c = i * SNBUF + s
            c_o = c - (SNBUF - 1)
            so = (s + 1) % SNBUF

            @pl.when(c < CHUNKS)
            def _():
                @pl.when(c >= SNBUF)
                def _():
                    pltpu.make_async_copy(
                        buf.at[s], acc.at[dv.at[c]], asem[s]).wait()
                pltpu.async_copy(
                    m_hbm.at[pl.ds(ebase + c * CW, CW)], buf.at[s], lsem[s])

            @pl.when((c_o >= 0) & (c_o < CHUNKS))
            def _():
                pltpu.make_async_copy(
                    m_hbm.at[pl.ds(ebase + c_o * CW, CW)], buf.at[so],
                    lsem[so]).wait()
                pltpu.async_copy(buf.at[so], acc.at[dv.at[c_o]], asem[so],
                                 add=True)
        return carry

    lax.fori_loop(0, CHUNKS // SNBUF + 1, body, 0)
    for s in range(SNBUF):
        pltpu.make_async_copy(buf.at[s], acc.at[dv.at[0]], asem[s]).wait()
    plsc.subcore_barrier()
    pltpu.sync_copy(acc.at[pl.ds(rbase, ROWS_PT)],
                    out_hbm.at[cid, pl.ds(rbase, ROWS_PT)])


def _lin_prep_body(x_ref, w_ref, b_ref, wa_ref, wb_ref,
                   h_ref, a_ref, bt_ref):
    h = jnp.dot(x_ref[...], w_ref[...],
                preferred_element_type=jnp.float32) + b_ref[...]
    h_ref[...] = h
    a_ref[...] = jnp.dot(h, wa_ref[...], preferred_element_type=jnp.float32)
    bt_ref[...] = jnp.dot(h, wb_ref[...], preferred_element_type=jnp.float32)


def _tc_lin_prep(x, w, b, wa, wb):
    return pl.pallas_call(
        _lin_prep_body,
        grid=(NBLK,),
        in_specs=[pl.BlockSpec((BLK, EMB), lambda i: (i, 0)),
                  pl.BlockSpec((EMB, EMB), lambda i: (0, 0)),
                  pl.BlockSpec((1, EMB), lambda i: (0, 0)),
                  pl.BlockSpec((EMB, EMB), lambda i: (0, 0)),
                  pl.BlockSpec((EMB, EMB), lambda i: (0, 0))],
        out_specs=[pl.BlockSpec((BLK, EMB), lambda i: (i, 0)),
                   pl.BlockSpec((BLK, EMB), lambda i: (i, 0)),
                   pl.BlockSpec((BLK, EMB), lambda i: (i, 0))],
        out_shape=[jax.ShapeDtypeStruct((N_PAD, EMB), jnp.float32),
                   jax.ShapeDtypeStruct((N_PAD, EMB), jnp.float32),
                   jax.ShapeDtypeStruct((N_PAD, EMB), jnp.float32)],
    )(x, w, b, wa, wb)


def _edge_body(p1_ref, p2_ref, ea_ref, wc_ref, b1_ref, w2_ref, b2_ref, m_ref):
    t = (p1_ref[...].astype(jnp.float32) + p2_ref[...].astype(jnp.float32)
         + jnp.dot(ea_ref[...], wc_ref[...], preferred_element_type=jnp.float32)
         + b1_ref[...])
    t = jnp.tanh(t)
    m_ref[...] = jnp.tanh(
        jnp.dot(t, w2_ref[...], preferred_element_type=jnp.float32) + b2_ref[...])


def _tc_edge(p1, p2, ea, wc, b1, w2, b2):
    return pl.pallas_call(
        _edge_body,
        grid=(EBLK,),
        in_specs=[pl.BlockSpec((BLK, EMB), lambda i: (i, 0)),
                  pl.BlockSpec((BLK, EMB), lambda i: (i, 0)),
                  pl.BlockSpec((BLK, ED), lambda i: (i, 0)),
                  pl.BlockSpec((ED, EMB), lambda i: (0, 0)),
                  pl.BlockSpec((1, EMB), lambda i: (0, 0)),
                  pl.BlockSpec((EMB, EMB), lambda i: (0, 0)),
                  pl.BlockSpec((1, EMB), lambda i: (0, 0))],
        out_specs=pl.BlockSpec((BLK, EMB), lambda i: (i, 0)),
        out_shape=jax.ShapeDtypeStruct((E_PAD, EMB), jnp.float32),
    )(p1, p2, ea, wc, b1, w2, b2)


def _upd_common(h_ref, par_ref, ua_ref, ub_ref, b1_ref, w2_ref, b2_ref):
    h = h_ref[...]
    aggr = par_ref[0] + par_ref[1]
    t = jnp.tanh(jnp.dot(h, ua_ref[...], preferred_element_type=jnp.float32)
                 + jnp.dot(aggr, ub_ref[...], preferred_element_type=jnp.float32)
                 + b1_ref[...])
    u = jnp.tanh(jnp.dot(t, w2_ref[...], preferred_element_type=jnp.float32)
                 + b2_ref[...])
    return h + u


def _upd_prep_body(h_ref, par_ref, ua_ref, ub_ref, b1_ref, w2_ref, b2_ref,
                   wa_ref, wb_ref, ho_ref, a_ref, bt_ref):
    hn = _upd_common(h_ref, par_ref, ua_ref, ub_ref, b1_ref, w2_ref, b2_ref)
    ho_ref[...] = hn
    a_ref[...] = jnp.dot(hn, wa_ref[...], preferred_element_type=jnp.float32)
    bt_ref[...] = jnp.dot(hn, wb_ref[...], preferred_element_type=jnp.float32)


def _tc_update_prep(h, par, ua, ub, b1, w2, b2, wa, wb):
    wspec = pl.BlockSpec((EMB, EMB), lambda i: (0, 0))
    bspec = pl.BlockSpec((1, EMB), lambda i: (0, 0))
    nspec = pl.BlockSpec((BLK, EMB), lambda i: (i, 0))
    return pl.pallas_call(
        _upd_prep_body,
        grid=(NBLK,),
        in_specs=[nspec,
                  pl.BlockSpec((NC, BLK, EMB), lambda i: (0, i, 0)),
                  wspec, wspec, bspec, wspec, bspec, wspec, wspec],
        out_specs=[nspec, nspec, nspec],
        out_shape=[jax.ShapeDtypeStruct((N_PAD, EMB), jnp.float32),
                   jax.ShapeDtypeStruct((N_PAD, EMB), jnp.float32),
                   jax.ShapeDtypeStruct((N_PAD, EMB), jnp.float32)],
    )(h, par, ua, ub, b1, w2, b2, wa, wb)


def _upd_pool_body(h_ref, par_ref, ua_ref, ub_ref, b1_ref, w2_ref, b2_ref,
                   bat_ref, o_ref, sums, counts):
    i = pl.program_id(0)

    @pl.when(i == 0)
    def _():
        sums[...] = jnp.zeros_like(sums)
        counts[...] = jnp.zeros_like(counts)

    hn = _upd_common(h_ref, par_ref, ua_ref, ub_ref, b1_ref, w2_ref, b2_ref)
    bat = bat_ref[0]
    gids = lax.broadcasted_iota(jnp.int32, (G, BLK), 0)
    onehot = jnp.where(bat == gids, 1.0, 0.0)
    sums[...] += jnp.dot(onehot, hn, preferred_element_type=jnp.float32)
    counts[...] += jnp.sum(onehot, axis=1, keepdims=True)

    @pl.when(i == NBLK - 1)
    def _():
        o_ref[...] = sums[...] / jnp.maximum(counts[...], 1.0)


def _tc_update_pool(h, par, ua, ub, b1, w2, b2, bat3):
    wspec = pl.BlockSpec((EMB, EMB), lambda i: (0, 0))
    bspec = pl.BlockSpec((1, EMB), lambda i: (0, 0))
    return pl.pallas_call(
        _upd_pool_body,
        grid=(NBLK,),
        in_specs=[pl.BlockSpec((BLK, EMB), lambda i: (i, 0)),
                  pl.BlockSpec((NC, BLK, EMB), lambda i: (0, i, 0)),
                  wspec, wspec, bspec, wspec, bspec,
                  pl.BlockSpec((1, 1, BLK), lambda i: (i, 0, 0))],
        out_specs=pl.BlockSpec((G, EMB), lambda i: (0, 0)),
        out_shape=jax.ShapeDtypeStruct((G, EMB), jnp.float32),
        scratch_shapes=[pltpu.VMEM((G, EMB), jnp.float32),
                        pltpu.VMEM((G, 1), jnp.float32)],
    )(h, par, ua, ub, b1, w2, b2, bat3)


@jax.jit
def _impl(x, edge_index, edge_attr, batch, lin_W, lin_b,
          msg_W1, msg_b1, msg_W2, msg_b2,
          upd_W1, upd_b1, upd_W2, upd_b2):
    f32 = jnp.float32
    x_p = jnp.zeros((N_PAD, EMB), f32).at[:N].set(x)
    src = edge_index[0]
    dst = edge_index[1]
    src_flat = jnp.zeros((E_PAD,), jnp.int32).at[:E].set(src)
    dst_flat = jnp.full((E_PAD,), N, jnp.int32).at[:E].set(dst)
    src_g = src_flat.reshape(NS, TCHUNKS, CW)
    dst_g = dst_flat.reshape(NS, TCHUNKS, CW)
    dst_p = dst_flat.reshape(NW, CHUNKS, CW)
    ea_p = jnp.zeros((E_PAD, ED), f32).at[:E].set(edge_attr)
    bat3 = jnp.full((N_PAD,), -1, jnp.int32).at[:N].set(batch).reshape(NBLK, 1, BLK)
    zrows = jnp.zeros((ROWS_PT, EMB), f32)

    h, a_tab, b_tab = _tc_lin_prep(x_p, lin_W, lin_b.reshape(1, EMB),
                                   msg_W1[0, :EMB], msg_W1[0, EMB:2 * EMB])
    for l in range(L):
        w1c = msg_W1[l, 2 * EMB:]
        p1, p2 = _sc_gather(a_tab, b_tab, dst_g, src_g)
        m = _tc_edge(p1, p2, ea_p, w1c,
                     msg_b1[l].reshape(1, EMB),
                     msg_W2[l], msg_b2[l].reshape(1, EMB))
        par = _sc_scatter(m, dst_p, zrows)
        if l < L - 1:
            h, a_tab, b_tab = _tc_update_prep(
                h, par, upd_W1[l, :EMB], upd_W1[l, EMB:],
                upd_b1[l].reshape(1, EMB), upd_W2[l],
                upd_b2[l].reshape(1, EMB),
                msg_W1[l + 1, :EMB], msg_W1[l + 1, EMB:2 * EMB])
        else:
            out = _tc_update_pool(
                h, par, upd_W1[l, :EMB], upd_W1[l, EMB:],
                upd_b1[l].reshape(1, EMB), upd_W2[l],
                upd_b2[l].reshape(1, EMB), bat3)
    return out


def kernel(x, edge_index, edge_attr, batch, lin_W, lin_b,
           msg_W1, msg_b1, msg_W2, msg_b2,
           upd_W1, upd_b1, upd_W2, upd_b2):
    return _impl(x, edge_index, edge_attr, batch, lin_W, lin_b,
                 msg_W1, msg_b1, msg_W2, msg_b2,
                 upd_W1, upd_b1, upd_W2, upd_b2)


# speedup vs baseline: 1.7659x; 1.0012x over previous
import functools

import jax
import jax.numpy as jnp
from jax import lax
from jax.experimental import pallas as pl
from jax.experimental.pallas import tpu as pltpu
from jax.experimental.pallas import tpu_sc as plsc

L = 4
N = 10000
E = 160000
EMB = 128
ED = 4
G = 16

NC = 2
NS = 16
NW = NC * NS
CW = 128
CHUNKS = 40
E_PAD = NW * CHUNKS * CW
BLK = 512
N_PAD = 10240
ROWS_PT = N_PAD // NS
NBLK = N_PAD // BLK
EBLK = E_PAD // BLK

_mesh = plsc.VectorSubcoreMesh(core_axis_name="c", subcore_axis_name="s")


GK = 1
GNB = 6
TCHUNKS = E_PAD // NS // CW
BATCHES = TCHUNKS // GK
BROWS = GK * CW


def _gather_pipe(tab_hbm, idx_hbm, out_hbm, iv, bufs, gsem, osem, tid):
    pltpu.sync_copy(idx_hbm.at[tid], iv)
    ebase = tid * (TCHUNKS * CW)

    def body(i, carry):
        for s in range(GNB):
            b = i * GNB + s
            b_o = b - (GNB - 1)
            so = (s + 1) % GNB

            @pl.when(b < BATCHES)
            def _():
                @pl.when(b >= GNB)
                def _():
                    pltpu.make_async_copy(
                        bufs.at[s], out_hbm.at[pl.ds(0, BROWS)], osem[s]).wait()
                for k in range(GK):
                    pltpu.async_copy(tab_hbm.at[iv.at[b * GK + k]],
                                     bufs.at[s, pl.ds(k * CW, CW)], gsem[s])

            @pl.when((b_o >= 0) & (b_o < BATCHES))
            def _():
                for k in range(GK):
                    pltpu.make_async_copy(
                        tab_hbm.at[iv.at[b_o * GK + k]],
                        bufs.at[so, pl.ds(k * CW, CW)], gsem[so]).wait()
                off = ebase + b_o * BROWS
                pltpu.async_copy(bufs.at[so], out_hbm.at[pl.ds(off, BROWS)],
                                 osem[so])
        return carry

    lax.fori_loop(0, BATCHES // GNB + 2, body, 0)
    for s in range(GNB):
        pltpu.make_async_copy(bufs.at[s], out_hbm.at[pl.ds(0, BROWS)],
                              osem[s]).wait()


@functools.partial(
    pl.kernel,
    out_type=(jax.ShapeDtypeStruct((E_PAD, EMB), jnp.float32),
              jax.ShapeDtypeStruct((E_PAD, EMB), jnp.float32)),
    mesh=_mesh,
    scratch_types=[
        pltpu.VMEM((TCHUNKS, CW), jnp.int32),
        pltpu.VMEM((GNB, BROWS, EMB), jnp.float32),
    ] + [pltpu.SemaphoreType.DMA] * (2 * GNB),
)
def _sc_gather(a_hbm, b_hbm, dstg_hbm, srcg_hbm, p1_hbm, p2_hbm,
               iv, bufs, *sems):
    gsem = sems[0:GNB]
    osem = sems[GNB:2 * GNB]
    cid = lax.axis_index("c")
    tid = lax.axis_index("s")

    @pl.when(cid == 0)
    def _():
        _gather_pipe(a_hbm, dstg_hbm, p1_hbm, iv, bufs, gsem, osem, tid)

    @pl.when(cid == 1)
    def _():
        _gather_pipe(b_hbm, srcg_hbm, p2_hbm, iv, bufs, gsem, osem, tid)


SNBUF = 2


@functools.partial(
    pl.kernel,
    out_type=jax.ShapeDtypeStruct((NC, N_PAD, EMB), jnp.float32),
    mesh=_mesh,
    scratch_types=[
        pltpu.VMEM((CHUNKS, CW), jnp.int32),
        pltpu.VMEM((SNBUF, CW, EMB), jnp.float32),
        pltpu.VMEM_SHARED((N_PAD, EMB), jnp.float32),
    ] + [pltpu.SemaphoreType.DMA] * (2 * SNBUF),
)
def _sc_scatter(m_hbm, dst_hbm, z_hbm, out_hbm, dv, buf, acc, *sems):
    lsem = sems[0:SNBUF]
    asem = sems[SNBUF:2 * SNBUF]
    cid = lax.axis_index("c")
    sid = lax.axis_index("s")
    wid = sid * NC + cid
    rbase = sid * ROWS_PT
    ebase = wid * (CHUNKS * CW)
    for s in range(SNBUF):
        pltpu.async_copy(
            m_hbm.at[pl.ds(ebase + s * CW, CW)], buf.at[s], lsem[s])
    pltpu.sync_copy(dst_hbm.at[wid], dv)
    pltpu.sync_copy(z_hbm, acc.at[pl.ds(rbase, ROWS_PT)])
    plsc.subcore_barrier()

    def body(i, carry):
        for s in range(SNBUF):
            c = i * SNBUF + s
            c_o = c - (SNBUF - 1)
            so = (s + 1) % SNBUF

            @pl.when((c >= SNBUF) & (c < CHUNKS))
            def _():
                pltpu.make_async_copy(
                    buf.at[s], acc.at[dv.at[c]], asem[s]).wait()
                pltpu.async_copy(
                    m_hbm.at[pl.ds(ebase + c * CW, CW)], buf.at[s], lsem[s])

            @pl.when((c_o >= 0) & (c_o < CHUNKS))
            def _():
                pltpu.make_async_copy(
                    m_hbm.at[pl.ds(ebase + c_o * CW, CW)], buf.at[so],
                    lsem[so]).wait()
                pltpu.async_copy(buf.at[so], acc.at[dv.at[c_o]], asem[so],
                                 add=True)
        return carry

    lax.fori_loop(0, CHUNKS // SNBUF + 1, body, 0)
    for s in range(SNBUF):
        pltpu.make_async_copy(buf.at[s], acc.at[dv.at[0]], asem[s]).wait()
    plsc.subcore_barrier()
    pltpu.sync_copy(acc.at[pl.ds(rbase, ROWS_PT)],
                    out_hbm.at[cid, pl.ds(rbase, ROWS_PT)])


def _lin_prep_body(x_ref, w_ref, b_ref, wa_ref, wb_ref,
                   h_ref, a_ref, bt_ref):
    h = jnp.dot(x_ref[...], w_ref[...],
                preferred_element_type=jnp.float32) + b_ref[...]
    h_ref[...] = h
    a_ref[...] = jnp.dot(h, wa_ref[...], preferred_element_type=jnp.float32)
    bt_ref[...] = jnp.dot(h, wb_ref[...], preferred_element_type=jnp.float32)


def _tc_lin_prep(x, w, b, wa, wb):
    return pl.pallas_call(
        _lin_prep_body,
        grid=(NBLK,),
        in_specs=[pl.BlockSpec((BLK, EMB), lambda i: (i, 0)),
                  pl.BlockSpec((EMB, EMB), lambda i: (0, 0)),
                  pl.BlockSpec((1, EMB), lambda i: (0, 0)),
                  pl.BlockSpec((EMB, EMB), lambda i: (0, 0)),
                  pl.BlockSpec((EMB, EMB), lambda i: (0, 0))],
        out_specs=[pl.BlockSpec((BLK, EMB), lambda i: (i, 0)),
                   pl.BlockSpec((BLK, EMB), lambda i: (i, 0)),
                   pl.BlockSpec((BLK, EMB), lambda i: (i, 0))],
        out_shape=[jax.ShapeDtypeStruct((N_PAD, EMB), jnp.float32),
                   jax.ShapeDtypeStruct((N_PAD, EMB), jnp.float32),
                   jax.ShapeDtypeStruct((N_PAD, EMB), jnp.float32)],
    )(x, w, b, wa, wb)


def _edge_body(p1_ref, p2_ref, ea_ref, wc_ref, b1_ref, w2_ref, b2_ref, m_ref):
    t = (p1_ref[...].astype(jnp.float32) + p2_ref[...].astype(jnp.float32)
         + jnp.dot(ea_ref[...], wc_ref[...], preferred_element_type=jnp.float32)
         + b1_ref[...])
    t = jnp.tanh(t)
    m_ref[...] = jnp.tanh(
        jnp.dot(t, w2_ref[...], preferred_element_type=jnp.float32) + b2_ref[...])


def _tc_edge(p1, p2, ea, wc, b1, w2, b2):
    return pl.pallas_call(
        _edge_body,
        grid=(EBLK,),
        in_specs=[pl.BlockSpec((BLK, EMB), lambda i: (i, 0)),
                  pl.BlockSpec((BLK, EMB), lambda i: (i, 0)),
                  pl.BlockSpec((BLK, ED), lambda i: (i, 0)),
                  pl.BlockSpec((ED, EMB), lambda i: (0, 0)),
                  pl.BlockSpec((1, EMB), lambda i: (0, 0)),
                  pl.BlockSpec((EMB, EMB), lambda i: (0, 0)),
                  pl.BlockSpec((1, EMB), lambda i: (0, 0))],
        out_specs=pl.BlockSpec((BLK, EMB), lambda i: (i, 0)),
        out_shape=jax.ShapeDtypeStruct((E_PAD, EMB), jnp.float32),
    )(p1, p2, ea, wc, b1, w2, b2)


def _upd_common(h_ref, par_ref, ua_ref, ub_ref, b1_ref, w2_ref, b2_ref):
    h = h_ref[...]
    aggr = par_ref[0] + par_ref[1]
    t = jnp.tanh(jnp.dot(h, ua_ref[...], preferred_element_type=jnp.float32)
                 + jnp.dot(aggr, ub_ref[...], preferred_element_type=jnp.float32)
                 + b1_ref[...])
    u = jnp.tanh(jnp.dot(t, w2_ref[...], preferred_element_type=jnp.float32)
                 + b2_ref[...])
    return h + u


def _upd_prep_body(h_ref, par_ref, ua_ref, ub_ref, b1_ref, w2_ref, b2_ref,
                   wa_ref, wb_ref, ho_ref, a_ref, bt_ref):
    hn = _upd_common(h_ref, par_ref, ua_ref, ub_ref, b1_ref, w2_ref, b2_ref)
    ho_ref[...] = hn
    a_ref[...] = jnp.dot(hn, wa_ref[...], preferred_element_type=jnp.float32)
    bt_ref[...] = jnp.dot(hn, wb_ref[...], preferred_element_type=jnp.float32)


def _tc_update_prep(h, par, ua, ub, b1, w2, b2, wa, wb):
    wspec = pl.BlockSpec((EMB, EMB), lambda i: (0, 0))
    bspec = pl.BlockSpec((1, EMB), lambda i: (0, 0))
    nspec = pl.BlockSpec((BLK, EMB), lambda i: (i, 0))
    return pl.pallas_call(
        _upd_prep_body,
        grid=(NBLK,),
        in_specs=[nspec,
                  pl.BlockSpec((NC, BLK, EMB), lambda i: (0, i, 0)),
                  wspec, wspec, bspec, wspec, bspec, wspec, wspec],
        out_specs=[nspec, nspec, nspec],
        out_shape=[jax.ShapeDtypeStruct((N_PAD, EMB), jnp.float32),
                   jax.ShapeDtypeStruct((N_PAD, EMB), jnp.float32),
                   jax.ShapeDtypeStruct((N_PAD, EMB), jnp.float32)],
    )(h, par, ua, ub, b1, w2, b2, wa, wb)


def _upd_pool_body(h_ref, par_ref, ua_ref, ub_ref, b1_ref, w2_ref, b2_ref,
                   bat_ref, o_ref, sums, counts):
    i = pl.program_id(0)

    @pl.when(i == 0)
    def _():
        sums[...] = jnp.zeros_like(sums)
        counts[...] = jnp.zeros_like(counts)

    hn = _upd_common(h_ref, par_ref, ua_ref, ub_ref, b1_ref, w2_ref, b2_ref)
    bat = bat_ref[0]
    gids = lax.broadcasted_iota(jnp.int32, (G, BLK), 0)
    onehot = jnp.where(bat == gids, 1.0, 0.0)
    sums[...] += jnp.dot(onehot, hn, preferred_element_type=jnp.float32)
    counts[...] += jnp.sum(onehot, axis=1, keepdims=True)

    @pl.when(i == NBLK - 1)
    def _():
        o_ref[...] = sums[...] / jnp.maximum(counts[...], 1.0)


def _tc_update_pool(h, par, ua, ub, b1, w2, b2, bat3):
    wspec = pl.BlockSpec((EMB, EMB), lambda i: (0, 0))
    bspec = pl.BlockSpec((1, EMB), lambda i: (0, 0))
    return pl.pallas_call(
        _upd_pool_body,
        grid=(NBLK,),
        in_specs=[pl.BlockSpec((BLK, EMB), lambda i: (i, 0)),
                  pl.BlockSpec((NC, BLK, EMB), lambda i: (0, i, 0)),
                  wspec, wspec, bspec, wspec, bspec,
                  pl.BlockSpec((1, 1, BLK), lambda i: (i, 0, 0))],
        out_specs=pl.BlockSpec((G, EMB), lambda i: (0, 0)),
        out_shape=jax.ShapeDtypeStruct((G, EMB), jnp.float32),
        scratch_shapes=[pltpu.VMEM((G, EMB), jnp.float32),
                        pltpu.VMEM((G, 1), jnp.float32)],
    )(h, par, ua, ub, b1, w2, b2, bat3)


@jax.jit
def _impl(x, edge_index, edge_attr, batch, lin_W, lin_b,
          msg_W1, msg_b1, msg_W2, msg_b2,
          upd_W1, upd_b1, upd_W2, upd_b2):
    f32 = jnp.float32
    x_p = jnp.zeros((N_PAD, EMB), f32).at[:N].set(x)
    src = edge_index[0]
    dst = edge_index[1]
    src_flat = jnp.zeros((E_PAD,), jnp.int32).at[:E].set(src)
    dst_flat = jnp.full((E_PAD,), N, jnp.int32).at[:E].set(dst)
    src_g = src_flat.reshape(NS, TCHUNKS, CW)
    dst_g = dst_flat.reshape(NS, TCHUNKS, CW)
    dst_p = dst_flat.reshape(NW, CHUNKS, CW)
    ea_p = jnp.zeros((E_PAD, ED), f32).at[:E].set(edge_attr)
    bat3 = jnp.full((N_PAD,), -1, jnp.int32).at[:N].set(batch).reshape(NBLK, 1, BLK)
    zrows = jnp.zeros((ROWS_PT, EMB), f32)

    h, a_tab, b_tab = _tc_lin_prep(x_p, lin_W, lin_b.reshape(1, EMB),
                                   msg_W1[0, :EMB], msg_W1[0, EMB:2 * EMB])
    for l in range(L):
        w1c = msg_W1[l, 2 * EMB:]
        p1, p2 = _sc_gather(a_tab, b_tab, dst_g, src_g)
        m = _tc_edge(p1, p2, ea_p, w1c,
                     msg_b1[l].reshape(1, EMB),
                     msg_W2[l], msg_b2[l].reshape(1, EMB))
        par = _sc_scatter(m, dst_p, zrows)
        if l < L - 1:
            h, a_tab, b_tab = _tc_update_prep(
                h, par, upd_W1[l, :EMB], upd_W1[l, EMB:],
                upd_b1[l].reshape(1, EMB), upd_W2[l],
                upd_b2[l].reshape(1, EMB),
                msg_W1[l + 1, :EMB], msg_W1[l + 1, EMB:2 * EMB])
        else:
            out = _tc_update_pool(
                h, par, upd_W1[l, :EMB], upd_W1[l, EMB:],
                upd_b1[l].reshape(1, EMB), upd_W2[l],
                upd_b2[l].reshape(1, EMB), bat3)
    return out


def kernel(x, edge_index, edge_attr, batch, lin_W, lin_b,
           msg_W1, msg_b1, msg_W2, msg_b2,
           upd_W1, upd_b1, upd_W2, upd_b2):
    return _impl(x, edge_index, edge_attr, batch, lin_W, lin_b,
                 msg_W1, msg_b1, msg_W2, msg_b2,
                 upd_W1, upd_b1, upd_W2, upd_b2)
